# baseline (speedup 1.0000x reference)
import functools

import jax
import jax.numpy as jnp
from jax import lax
from jax.experimental import pallas as pl
from jax.experimental.pallas import tpu as pltpu
from jax.experimental.pallas import tpu_sc as plsc

CHUNK = 128


def kernel(x, table):
    B, F = x.shape
    V, D = table.shape
    N = B * F
    assert N % CHUNK == 0
    n_chunks = N // CHUNK

    info = plsc.get_sparse_core_info()
    NC, NS = info.num_cores, info.num_subcores
    NW = NC * NS
    assert n_chunks % NW == 0
    cpw = n_chunks // NW

    idx2d = x.reshape(n_chunks, CHUNK).astype(jnp.int32)
    mesh = plsc.VectorSubcoreMesh(core_axis_name="c", subcore_axis_name="s")

    @functools.partial(
        pl.kernel,
        mesh=mesh,
        compiler_params=pltpu.CompilerParams(use_tc_tiling_on_sc=False),
        out_type=jax.ShapeDtypeStruct((N, D), jnp.float32),
        scratch_types=[
            pltpu.VMEM((cpw, CHUNK), jnp.int32),
            pltpu.VMEM((CHUNK, D), jnp.float32),
            pltpu.SemaphoreType.DMA,
        ],
    )
    def emb(idx_hbm, table_hbm, out_hbm, idx_v, rows_v, sem):
        wid = lax.axis_index("s") * NC + lax.axis_index("c")
        c0 = wid * cpw
        pltpu.sync_copy(idx_hbm.at[pl.ds(c0 * 1, cpw)], idx_v)

        def body(j, carry):
            pltpu.async_copy(table_hbm.at[idx_v.at[j]], rows_v, sem).wait()
            pltpu.sync_copy(rows_v, out_hbm.at[pl.ds((c0 + j) * CHUNK, CHUNK)])
            return carry

        lax.fori_loop(0, cpw, body, 0)

    out = emb(idx2d, table)
    return out.reshape(B, F, D)


# speedup vs baseline: 1.0778x; 1.0778x over previous
import functools

import jax
import jax.numpy as jnp
from jax import lax
from jax.experimental import pallas as pl
from jax.experimental.pallas import tpu as pltpu
from jax.experimental.pallas import tpu_sc as plsc

CHUNK = 128
RING = 8
AHEAD = 4


def kernel(x, table):
    B, F = x.shape
    V, D = table.shape
    N = B * F
    assert N % CHUNK == 0
    n_chunks = N // CHUNK

    info = plsc.get_sparse_core_info()
    NC, NS = info.num_cores, info.num_subcores
    NW = NC * NS
    assert n_chunks % NW == 0
    cpw = n_chunks // NW
    assert cpw % RING == 0

    idx2d = x.reshape(n_chunks, CHUNK).astype(jnp.int32)
    mesh = plsc.VectorSubcoreMesh(core_axis_name="c", subcore_axis_name="s")

    @functools.partial(
        pl.kernel,
        mesh=mesh,
        compiler_params=pltpu.CompilerParams(use_tc_tiling_on_sc=False),
        out_type=jax.ShapeDtypeStruct((N, D), jnp.float32),
        scratch_types=[
            pltpu.VMEM((cpw, CHUNK), jnp.int32),
        ]
        + [pltpu.VMEM((CHUNK, D), jnp.float32) for _ in range(RING)]
        + [pltpu.SemaphoreType.DMA for _ in range(2 * RING)],
    )
    def emb(idx_hbm, table_hbm, out_hbm, idx_v, *bufs):
        rows = bufs[:RING]
        gsem = bufs[RING:2 * RING]
        wsem = bufs[2 * RING:3 * RING]
        wid = lax.axis_index("s") * NC + lax.axis_index("c")
        c0 = wid * cpw
        pltpu.sync_copy(idx_hbm.at[pl.ds(c0, cpw)], idx_v)

        for b in range(AHEAD):
            pltpu.async_copy(table_hbm.at[idx_v.at[b]], rows[b], gsem[b])

        def body(i, carry):
            j0 = i * RING
            for b in range(RING):
                j = j0 + b
                jn = j + AHEAD
                bn = (b + AHEAD) % RING

                @pl.when(jnp.logical_and(jn < cpw, jn >= RING))
                def _drain_write():
                    pltpu.make_async_copy(
                        rows[bn], out_hbm.at[pl.ds(0, CHUNK)], wsem[bn]
                    ).wait()

                @pl.when(jn < cpw)
                def _fire_gather():
                    pltpu.async_copy(
                        table_hbm.at[idx_v.at[jn]], rows[bn], gsem[bn]
                    )

                pltpu.make_async_copy(
                    table_hbm.at[idx_v.at[j]], rows[b], gsem[b]
                ).wait()
                pltpu.async_copy(
                    rows[b], out_hbm.at[pl.ds((c0 + j) * CHUNK, CHUNK)], wsem[b]
                )
            return carry

        lax.fori_loop(0, cpw // RING, body, 0)

        for b in range(RING):
            pltpu.make_async_copy(
                rows[b], out_hbm.at[pl.ds(0, CHUNK)], wsem[b]
            ).wait()

    out = emb(idx2d, table)
    return out.reshape(B, F, D)


# speedup vs baseline: 1.1540x; 1.0708x over previous
import functools

import jax
import jax.numpy as jnp
from jax import lax
from jax.experimental import pallas as pl
from jax.experimental.pallas import tpu as pltpu
from jax.experimental.pallas import tpu_sc as plsc

BROW = 4
RING = 8
AHEAD = 4


def kernel(x, table):
    B, F = x.shape
    V, D = table.shape
    CHUNK = BROW * F
    assert B % BROW == 0
    n_chunks = B // BROW

    info = plsc.get_sparse_core_info()
    NC, NS = info.num_cores, info.num_subcores
    NW = NC * NS
    assert n_chunks % NW == 0
    cpw = n_chunks // NW
    assert cpw % RING == 0

    idx2d = (x.astype(jnp.int32) * 2).reshape(n_chunks, CHUNK)
    tpad = jnp.pad(table, ((0, 0), (0, 128 - D))).reshape(2 * V, D)
    mesh = plsc.VectorSubcoreMesh(core_axis_name="c", subcore_axis_name="s")

    @functools.partial(
        pl.kernel,
        mesh=mesh,
        compiler_params=pltpu.CompilerParams(use_tc_tiling_on_sc=False),
        out_type=jax.ShapeDtypeStruct((B, F, D), jnp.float32),
        scratch_types=[
            pltpu.VMEM((cpw, CHUNK), jnp.int32),
        ]
        + [pltpu.VMEM((CHUNK, D), jnp.float32) for _ in range(RING)]
        + [pltpu.SemaphoreType.DMA for _ in range(2 * RING)],
    )
    def emb(idx_hbm, table_hbm, out_hbm, idx_v, *bufs):
        rows = bufs[:RING]
        gsem = bufs[RING:2 * RING]
        wsem = bufs[2 * RING:3 * RING]
        wid = lax.axis_index("s") * NC + lax.axis_index("c")
        c0 = wid * cpw
        pltpu.sync_copy(idx_hbm.at[pl.ds(c0, cpw)], idx_v)

        def fire_writes(j, b):
            for k in range(BROW):
                pltpu.async_copy(
                    rows[b].at[pl.ds(k * F, F)],
                    out_hbm.at[BROW * (c0 + j) + k],
                    wsem[b],
                )

        def drain_writes(b):
            for k in range(BROW):
                pltpu.make_async_copy(
                    rows[b].at[pl.ds(k * F, F)], out_hbm.at[0], wsem[b]
                ).wait()

        for b in range(AHEAD):
            pltpu.async_copy(table_hbm.at[idx_v.at[b]], rows[b], gsem[b])

        def body(i, carry):
            j0 = i * RING
            for b in range(RING):
                j = j0 + b
                jn = j + AHEAD
                bn = (b + AHEAD) % RING

                @pl.when(jnp.logical_and(jn < cpw, jn >= RING))
                def _drain_write():
                    drain_writes(bn)

                @pl.when(jn < cpw)
                def _fire_gather():
                    pltpu.async_copy(
                        table_hbm.at[idx_v.at[jn]], rows[bn], gsem[bn]
                    )

                pltpu.make_async_copy(
                    table_hbm.at[idx_v.at[j]], rows[b], gsem[b]
                ).wait()
                fire_writes(j, b)
            return carry

        lax.fori_loop(0, cpw // RING, body, 0)

        for b in range(RING):
            drain_writes(b)

    return emb(idx2d, tpad)
